# baseline (speedup 1.0000x reference)
import functools

import jax
import jax.numpy as jnp
from jax import lax
from jax.experimental import pallas as pl
from jax.experimental.pallas import tpu as pltpu
from jax.experimental.pallas import tpu_sc as plsc

NUM_CORES = 2
NUM_SUBCORES = 16
NUM_WORKERS = NUM_CORES * NUM_SUBCORES
LANES = 16
FACTORS = 64
CHUNK = 128


def _bpr_body(user_hbm, item_i_hbm, item_j_hbm,
              user_table, item_table, bias_hbm, out_hbm,
              uidx_v, iidx_v, jidx_v,
              u_rows, i_rows, j_rows, bi_v, bj_v, out_v, sem):
    b_per_w = uidx_v.shape[0]
    n_chunks = b_per_w // CHUNK
    wid = lax.axis_index("s") * NUM_CORES + lax.axis_index("c")
    base = wid * b_per_w

    pltpu.sync_copy(user_hbm.at[pl.ds(base, b_per_w)], uidx_v)
    pltpu.sync_copy(item_i_hbm.at[pl.ds(base, b_per_w)], iidx_v)
    pltpu.sync_copy(item_j_hbm.at[pl.ds(base, b_per_w)], jidx_v)

    copies = []
    for k in range(n_chunks):
        sl = pl.ds(k * CHUNK, CHUNK)
        copies.append(pltpu.async_copy(
            user_table.at[uidx_v.at[sl]], u_rows.at[sl], sem))
        copies.append(pltpu.async_copy(
            item_table.at[iidx_v.at[sl]], i_rows.at[sl], sem))
        copies.append(pltpu.async_copy(
            item_table.at[jidx_v.at[sl]], j_rows.at[sl], sem))
        copies.append(pltpu.async_copy(
            bias_hbm.at[iidx_v.at[sl]], bi_v.at[sl], sem))
        copies.append(pltpu.async_copy(
            bias_hbm.at[jidx_v.at[sl]], bj_v.at[sl], sem))
    for c in copies:
        c.wait()

    iota16 = lax.iota(jnp.int32, LANES)

    def group_body(g, _):
        rbase = g * LANES
        rows = rbase + iota16
        acc = bi_v[pl.ds(rbase, LANES)] - bj_v[pl.ds(rbase, LANES)]
        for f in range(FACTORS):
            fvec = jnp.full((LANES,), f, jnp.int32)
            u = plsc.load_gather(u_rows, [rows, fvec])
            iv = plsc.load_gather(i_rows, [rows, fvec])
            jv = plsc.load_gather(j_rows, [rows, fvec])
            acc = acc + u * (iv - jv)
        out_v[pl.ds(rbase, LANES)] = acc
        return 0

    lax.fori_loop(0, b_per_w // LANES, group_body, 0)

    pltpu.sync_copy(out_v, out_hbm.at[pl.ds(base, b_per_w)])


def _make_kernel(batch):
    b_per_w = batch // NUM_WORKERS
    mesh = plsc.VectorSubcoreMesh(core_axis_name="c", subcore_axis_name="s")
    return pl.kernel(
        _bpr_body,
        mesh=mesh,
        compiler_params=pltpu.CompilerParams(
            needs_layout_passes=False, use_tc_tiling_on_sc=False),
        out_type=jax.ShapeDtypeStruct((batch,), jnp.float32),
        scratch_types=[
            pltpu.VMEM((b_per_w,), jnp.int32),
            pltpu.VMEM((b_per_w,), jnp.int32),
            pltpu.VMEM((b_per_w,), jnp.int32),
            pltpu.VMEM((b_per_w, FACTORS), jnp.float32),
            pltpu.VMEM((b_per_w, FACTORS), jnp.float32),
            pltpu.VMEM((b_per_w, FACTORS), jnp.float32),
            pltpu.VMEM((b_per_w,), jnp.float32),
            pltpu.VMEM((b_per_w,), jnp.float32),
            pltpu.VMEM((b_per_w,), jnp.float32),
            pltpu.SemaphoreType.DMA,
        ],
    )


def kernel(user, item_i, item_j, user_table, item_table, item_bias_table):
    batch = user.shape[0]
    fn = _make_kernel(batch)
    return fn(user.astype(jnp.int32), item_i.astype(jnp.int32),
              item_j.astype(jnp.int32), user_table, item_table,
              item_bias_table.reshape(-1))
